# baseline (speedup 1.0000x reference)
import jax
import jax.numpy as jnp
from jax.experimental import pallas as pl
from jax.experimental.pallas import tpu as pltpu


def _tc_kernel(B, F, h, w, b_blk, interpret=False):
    hw = h * w

    def body(row_ref, col_ref, out_ref, pos_ref):
        @pl.when(pl.program_id(0) == 0)
        def _():
            colT = col_ref[...].T
            rowT = row_ref[...].T
            top = jnp.tile(colT, (1, h))
            bot = jnp.repeat(rowT, w, axis=1)
            pos_ref[...] = jnp.concatenate([top, bot], axis=0)

        out_ref[...] = jnp.broadcast_to(pos_ref[...][None], (b_blk, 2 * F, hw))

    return pl.pallas_call(
        body,
        grid=(B // b_blk,),
        in_specs=[
            pl.BlockSpec((h, F), lambda i: (0, 0)),
            pl.BlockSpec((w, F), lambda i: (0, 0)),
        ],
        out_specs=pl.BlockSpec((b_blk, 2 * F, hw), lambda i: (i, 0, 0)),
        out_shape=jax.ShapeDtypeStruct((B, 2 * F, hw), jnp.float32),
        scratch_shapes=[pltpu.VMEM((2 * F, hw), jnp.float32)],
        interpret=interpret,
    )


def kernel(token_tensors, row_embed_w, col_embed_w):
    B, _, h, w = token_tensors.shape
    F = row_embed_w.shape[1]
    out = _tc_kernel(B, F, h, w, b_blk=4)(row_embed_w, col_embed_w)
    return out.reshape(B, 2 * F, h, w)
